# baseline (speedup 1.0000x reference)
import jax
import jax.numpy as jnp
from jax.experimental import pallas as pl
from jax.experimental.pallas import tpu as pltpu

_N = 32768
_D = 256
_DOUT = 128
_NSEG = 16
_NLAYERS = 3
_B = 4096
_NB = _N // _B
_EPS = 1e-5
_F32 = jnp.float32
_BF16 = jnp.bfloat16


def _bdot(a, b):
    return jnp.dot(a.astype(_BF16), b.astype(_BF16),
                   preferred_element_type=_F32)


def _mlp_rows(x, w1, b1, w2, b2):
    h = jnp.maximum(_bdot(x, w1) + b1, 0.0)
    return _bdot(h, w2) + b2


def _onehot_t(seg_ref):
    sv = seg_ref[0]
    ids = jax.lax.broadcasted_iota(jnp.int32, (_NSEG, _B), 0)
    ot = jnp.where(ids == sv, 1.0, 0.0).astype(_F32)
    return ot.astype(_BF16)


def _split(v):
    hi = v.astype(_BF16)
    lo = (v - hi.astype(_F32)).astype(_BF16)
    return hi, lo


def _otdot(ot, v):
    hi, lo = _split(v)
    return (jnp.dot(ot, lo, preferred_element_type=_F32) +
            jnp.dot(ot, hi, preferred_element_type=_F32))


def _otdot_hi(ot, v):
    return jnp.dot(ot, v.astype(_BF16), preferred_element_type=_F32)


def _apply_bn(h_ref, xphi_ref, ot, rhi, rlo, scale, shift):
    dn = (((0,), (0,)), ((), ()))
    rr = (jax.lax.dot_general(ot, rlo[...], dn, preferred_element_type=_F32) +
          jax.lax.dot_general(ot, rhi[...], dn, preferred_element_type=_F32))
    t = xphi_ref[...].astype(_F32) + rr
    return h_ref[...] + t * scale[...] + shift[...]


def _first_kernel(x_ref, seg_ref, w1, b1, w2, b2,
                  xphi_out, ssh_out, ssp_out, sq_out, cnt_out):
    ot = _onehot_t(seg_ref)
    x = x_ref[...]
    xp = _mlp_rows(x, w1[...], b1[...], w2[...], b2[...])
    xphi_out[...] = xp.astype(_BF16)
    ssh_out[0] = _otdot(ot, x)
    ssp_out[0] = _otdot_hi(ot, xp)
    sq_out[0] = jnp.sum(xp * xp, axis=0, keepdims=True)
    cnt_out[0] = jnp.broadcast_to(
        jnp.sum(ot.astype(_F32), axis=1, keepdims=True), (_NSEG, 128))


def _mid_kernel(h_ref, xphi_ref, seg_ref,
                rhi, rlo, scale, shift,
                pw1, pb1, pw2, pb2,
                h_out, xphi_out, ssh_out, ssp_out, sq_out):
    ot = _onehot_t(seg_ref)
    hn = _apply_bn(h_ref, xphi_ref, ot, rhi, rlo, scale, shift)
    h_out[...] = hn
    xp = _mlp_rows(hn, pw1[...], pb1[...], pw2[...], pb2[...])
    xphi_out[...] = xp.astype(_BF16)
    ssh_out[0] = _otdot(ot, hn)
    ssp_out[0] = _otdot_hi(ot, xp)
    sq_out[0] = jnp.sum(xp * xp, axis=0, keepdims=True)


def _final_kernel(h_ref, xphi_ref, seg_ref,
                  rhi, rlo, scale, shift,
                  pw1, pb1, pw2, pb2,
                  ssp_out):
    ot = _onehot_t(seg_ref)
    hn = _apply_bn(h_ref, xphi_ref, ot, rhi, rlo, scale, shift)
    xp = _mlp_rows(hn, pw1[...], pb1[...], pw2[...], pb2[...])
    ssp_out[0] = _otdot(ot, xp)


def _tiny_kernel(ssh_p, ssp_p, sq_p, cnt_p,
                 rw1, rb1, rw2, rb2, bng, bnb,
                 rhi_out, rlo_out, scale_out, shift_out):
    s = jnp.sum(ssh_p[...], axis=0)
    r = _mlp_rows(s, rw1[...], rb1[...], rw2[...], rb2[...])
    c = jnp.sum(cnt_p[...], axis=0)[:, :1]
    g = jnp.sum(ssp_p[...], axis=0)
    s1 = jnp.sum(g + c * r, axis=0, keepdims=True)
    s2 = (jnp.sum(sq_p[...], axis=0) +
          jnp.sum((2.0 * g + c * r) * r, axis=0, keepdims=True))
    mean = s1 / _N
    var = s2 / _N - mean * mean
    sc = bng[...] / jnp.sqrt(var + _EPS)
    rhi, rlo = _split(r)
    rhi_out[...] = rhi
    rlo_out[...] = rlo
    scale_out[...] = sc
    shift_out[...] = bnb[...] - mean * sc


def _out_kernel(ssp_p, qw1, qb1, qw2, qb2, out_ref):
    s = jnp.sum(ssp_p[...], axis=0)
    out_ref[...] = _mlp_rows(s, qw1[...], qb1[...], qw2[...], qb2[...])


def _row_spec():
    return pl.BlockSpec((_B, _D), lambda i: (i, 0))


def _seg_spec():
    return pl.BlockSpec((1, 1, _B), lambda i: (i, 0, 0))


def _const_spec(shape):
    return pl.BlockSpec(shape, lambda i: tuple(0 for _ in shape))


def _blk3_spec(shape):
    return pl.BlockSpec((1,) + shape, lambda i: (i, 0, 0))


def _mlp_args(p):
    return (p["W1"], p["b1"].reshape(1, -1), p["W2"], p["b2"].reshape(1, -1))


def _mlp_specs():
    return [_const_spec((_D, _D)), _const_spec((1, _D)),
            _const_spec((_D, _D)), _const_spec((1, _D))]


_PAR = pltpu.CompilerParams(dimension_semantics=("parallel",))

_STAT_SHAPES = (
    jax.ShapeDtypeStruct((_NB, _NSEG, _D), _F32),
    jax.ShapeDtypeStruct((_NB, _NSEG, _D), _F32),
    jax.ShapeDtypeStruct((_NB, 1, _D), _F32),
)
_STAT_SPECS = (_blk3_spec((_NSEG, _D)), _blk3_spec((_NSEG, _D)),
               _blk3_spec((1, _D)))


def _first_pass(x, seg3, phi):
    out_shapes = (jax.ShapeDtypeStruct((_N, _D), _BF16),) + _STAT_SHAPES + (
        jax.ShapeDtypeStruct((_NB, _NSEG, 128), _F32),)
    out_specs = (_row_spec(),) + _STAT_SPECS + (_blk3_spec((_NSEG, 128)),)
    return pl.pallas_call(
        _first_kernel,
        grid=(_NB,),
        in_specs=[_row_spec(), _seg_spec()] + _mlp_specs(),
        out_specs=out_specs,
        out_shape=out_shapes,
        compiler_params=_PAR,
    )(x, seg3, *_mlp_args(phi))


def _small_specs():
    return [_const_spec((_NSEG, _D)), _const_spec((_NSEG, _D)),
            _const_spec((1, _D)), _const_spec((1, _D))]


def _mid_pass(h, xphi, seg3, small, phi_next):
    out_shapes = (
        jax.ShapeDtypeStruct((_N, _D), _F32),
        jax.ShapeDtypeStruct((_N, _D), _BF16),
    ) + _STAT_SHAPES
    out_specs = (_row_spec(), _row_spec()) + _STAT_SPECS
    return pl.pallas_call(
        _mid_kernel,
        grid=(_NB,),
        in_specs=([_row_spec(), _row_spec(), _seg_spec()]
                  + _small_specs() + _mlp_specs()),
        out_specs=out_specs,
        out_shape=out_shapes,
        compiler_params=_PAR,
    )(h, xphi, seg3, *small, *_mlp_args(phi_next))


def _final_pass(h, xphi, seg3, small, pool_phi):
    return pl.pallas_call(
        _final_kernel,
        grid=(_NB,),
        in_specs=([_row_spec(), _row_spec(), _seg_spec()]
                  + _small_specs() + _mlp_specs()),
        out_specs=_blk3_spec((_NSEG, _D)),
        out_shape=jax.ShapeDtypeStruct((_NB, _NSEG, _D), _F32),
        compiler_params=_PAR,
    )(h, xphi, seg3, *small, *_mlp_args(pool_phi))


def _tiny_pass(ssh_p, ssp_p, sq_p, cnt_p, rho, bng, bnb):
    out_shapes = (
        jax.ShapeDtypeStruct((_NSEG, _D), _BF16),
        jax.ShapeDtypeStruct((_NSEG, _D), _BF16),
        jax.ShapeDtypeStruct((1, _D), _F32),
        jax.ShapeDtypeStruct((1, _D), _F32),
    )
    return pl.pallas_call(
        _tiny_kernel,
        out_shape=out_shapes,
    )(ssh_p, ssp_p, sq_p, cnt_p, *_mlp_args(rho),
      bng.reshape(1, -1), bnb.reshape(1, -1))


def _out_pass(ssp_p, pool_rho):
    return pl.pallas_call(
        _out_kernel,
        out_shape=jax.ShapeDtypeStruct((_NSEG, _DOUT), _F32),
    )(ssp_p, *_mlp_args(pool_rho))


def kernel(x, seg, params):
    seg3 = seg.astype(jnp.int32).reshape(_NB, 1, _B)
    layers = params["layers"]
    xphi, ssh, ssp, sq, cnt = _first_pass(x, seg3, layers[0]["phi"])
    h = x
    for li in range(_NLAYERS):
        lyr = layers[li]
        small = _tiny_pass(ssh, ssp, sq, cnt,
                           lyr["rho"], lyr["bn_g"], lyr["bn_b"])
        if li < _NLAYERS - 1:
            h, xphi, ssh, ssp, sq = _mid_pass(
                h, xphi, seg3, small, layers[li + 1]["phi"])
        else:
            ssp_pool = _final_pass(h, xphi, seg3, small,
                                   params["pooling"]["phi"])
    return _out_pass(ssp_pool, params["pooling"]["rho"])
